# Initial kernel scaffold; baseline (speedup 1.0000x reference)
import jax
import jax.numpy as jnp
from jax import lax
from jax.experimental import pallas as pl
from jax.experimental.pallas import tpu as pltpu
from jax.experimental.pallas import tpu_sc as plsc

N = 10000
E = 320000
D = 128
G = 128
NC, NS, L = 2, 16, 16
NW = NC * NS
EPW = E // NW
CH = 128
NCHUNK = 80
EPAD = NCHUNK * CH
BLK = 1000
NBLK = N // BLK
NZCH = (N + CH - 1) // CH


def _pre_body(x_ref, w_ref, att2_ref, h_ref, as_ref, ad_ref):
    h = jnp.dot(x_ref[...], w_ref[...], preferred_element_type=jnp.float32)
    h_ref[...] = h
    asd = jnp.dot(h, att2_ref[...], preferred_element_type=jnp.float32)
    as_ref[...] = asd[:, 0:1]
    ad_ref[...] = asd[:, 1:2]


_pre = pl.pallas_call(
    _pre_body,
    grid=(NBLK,),
    in_specs=[
        pl.BlockSpec((BLK, D), lambda i: (i, 0)),
        pl.BlockSpec((D, D), lambda i: (0, 0)),
        pl.BlockSpec((D, 2), lambda i: (0, 0)),
    ],
    out_specs=[
        pl.BlockSpec((BLK, D), lambda i: (i, 0)),
        pl.BlockSpec((BLK, 1), lambda i: (i, 0)),
        pl.BlockSpec((BLK, 1), lambda i: (i, 0)),
    ],
    out_shape=[
        jax.ShapeDtypeStruct((N, D), jnp.float32),
        jax.ShapeDtypeStruct((N, 1), jnp.float32),
        jax.ShapeDtypeStruct((N, 1), jnp.float32),
    ],
)


def _sc_body(h_hbm, as_hbm, ad_hbm, src_hbm, dst_hbm,
             num_out, den_out,
             src_v, dst_v, as_v, ad_v, ex_v, rows_v, zbuf_v,
             num_sh, den_sh, sem):
    cid = lax.axis_index("c")
    sid = lax.axis_index("s")
    wid = sid * NC + cid
    zeros16 = jnp.zeros((L,), jnp.float32)

    def _zrow(r, carry):
        for k in range(D // L):
            zbuf_v[r, pl.ds(k * L, L)] = zeros16
        return carry
    lax.fori_loop(0, CH, _zrow, 0)

    for k in range(5):
        c = sid + k * NS

        @pl.when(c < NZCH - 1)
        def _():
            pltpu.sync_copy(zbuf_v, num_sh.at[pl.ds(c * CH, CH)])
            pltpu.sync_copy(zbuf_v.at[0], den_sh.at[pl.ds(c * CH, CH)])

        @pl.when(c == NZCH - 1)
        def _():
            rem = N - (NZCH - 1) * CH
            pltpu.sync_copy(zbuf_v.at[pl.ds(0, rem)],
                            num_sh.at[pl.ds((NZCH - 1) * CH, rem)])
            pltpu.sync_copy(zbuf_v.at[0, pl.ds(0, rem)],
                            den_sh.at[pl.ds((NZCH - 1) * CH, rem)])
    plsc.subcore_barrier()

    pltpu.sync_copy(src_hbm.at[wid], src_v)
    pltpu.sync_copy(dst_hbm.at[wid], dst_v)
    pltpu.sync_copy(as_hbm, as_v)
    pltpu.sync_copy(ad_hbm, ad_v)

    iota16 = lax.iota(jnp.int32, L)

    def _chunk(c, carry):
        for j in range(CH // L):
            s16 = src_v[c, pl.ds(j * L, L)]
            d16 = dst_v[c, pl.ds(j * L, L)]
            a_s = plsc.load_gather(as_v, [s16])
            a_d = plsc.load_gather(ad_v, [d16])
            e = a_s + a_d
            e = jnp.where(e > 0.0, e, 0.2 * e)
            ex = jnp.exp(e)
            valid = (c * CH + j * L + iota16) < EPW
            ex_v[c, pl.ds(j * L, L)] = jnp.where(valid, ex, 0.0)
        pltpu.sync_copy(ex_v.at[c], den_sh.at[dst_v.at[c]], add=True)
        pltpu.async_copy(h_hbm.at[src_v.at[c]], rows_v, sem).wait()

        def _scale(r, carry2):
            b = jnp.full((L,), ex_v[c, r], jnp.float32)
            for k in range(D // L):
                rows_v[r, pl.ds(k * L, L)] = rows_v[r, pl.ds(k * L, L)] * b
            return carry2
        lax.fori_loop(0, CH, _scale, 0)
        pltpu.sync_copy(rows_v, num_sh.at[dst_v.at[c]], add=True)
        return carry
    lax.fori_loop(0, NCHUNK, _chunk, 0)

    plsc.subcore_barrier()

    rpt = N // NS
    pltpu.sync_copy(num_sh.at[pl.ds(sid * rpt, rpt)],
                    num_out.at[cid, pl.ds(sid * rpt, rpt)])

    @pl.when(sid == 0)
    def _():
        pltpu.sync_copy(den_sh, den_out.at[cid])


_sc_edges = pl.kernel(
    _sc_body,
    out_type=[
        jax.ShapeDtypeStruct((NC, N, D), jnp.float32),
        jax.ShapeDtypeStruct((NC, N), jnp.float32),
    ],
    mesh=plsc.VectorSubcoreMesh(core_axis_name="c", subcore_axis_name="s"),
    scratch_types=[
        pltpu.VMEM((NCHUNK, CH), jnp.int32),
        pltpu.VMEM((NCHUNK, CH), jnp.int32),
        pltpu.VMEM((N,), jnp.float32),
        pltpu.VMEM((N,), jnp.float32),
        pltpu.VMEM((NCHUNK, CH), jnp.float32),
        pltpu.VMEM((CH, D), jnp.float32),
        pltpu.VMEM((CH, D), jnp.float32),
        pltpu.VMEM_SHARED((N, D), jnp.float32),
        pltpu.VMEM_SHARED((N,), jnp.float32),
        pltpu.SemaphoreType.DMA,
    ],
)


def _post_body(num_ref, den_ref, batch_ref, bias_ref, wf_ref, bf_ref,
               out_ref, sums_ref, counts_ref):
    i = pl.program_id(0)

    @pl.when(i == 0)
    def _():
        sums_ref[...] = jnp.zeros_like(sums_ref)
        counts_ref[...] = jnp.zeros_like(counts_ref)

    num = num_ref[0] + num_ref[1]
    den = den_ref[...][:, 0] + den_ref[...][:, 1]
    safe = jnp.where(den > 0.0, den, 1.0)
    out = jnp.where(den[:, None] > 0.0, num / safe[:, None], 0.0)
    out = out + bias_ref[...]
    g = out * jax.nn.sigmoid(out)
    b = batch_ref[0, 0, :]
    cols = lax.broadcasted_iota(jnp.int32, (BLK, G), 1)
    p = (b[:, None] == cols).astype(jnp.float32)
    sums_ref[...] += lax.dot_general(
        p, g, (((0,), (0,)), ((), ())), preferred_element_type=jnp.float32)
    counts_ref[...] += lax.dot_general(
        p, jnp.ones((BLK, 1), jnp.float32), (((0,), (0,)), ((), ())),
        preferred_element_type=jnp.float32)

    @pl.when(i == NBLK - 1)
    def _():
        mean = sums_ref[...] / jnp.maximum(counts_ref[...], 1.0)
        out_ref[...] = jnp.dot(mean, wf_ref[...],
                               preferred_element_type=jnp.float32) + bf_ref[...]


_post = pl.pallas_call(
    _post_body,
    grid=(NBLK,),
    in_specs=[
        pl.BlockSpec((NC, BLK, D), lambda i: (0, i, 0)),
        pl.BlockSpec((BLK, NC), lambda i: (i, 0)),
        pl.BlockSpec((1, 1, BLK), lambda i: (i, 0, 0)),
        pl.BlockSpec((1, D), lambda i: (0, 0)),
        pl.BlockSpec((D, 1), lambda i: (0, 0)),
        pl.BlockSpec((1, 1), lambda i: (0, 0)),
    ],
    out_specs=pl.BlockSpec((G, 1), lambda i: (0, 0)),
    out_shape=jax.ShapeDtypeStruct((G, 1), jnp.float32),
    scratch_shapes=[
        pltpu.VMEM((G, D), jnp.float32),
        pltpu.VMEM((G, 1), jnp.float32),
    ],
)


def kernel(x, edge_index, batch, W, att_src, att_dst, bias, Wf, bf):
    src = edge_index[0].astype(jnp.int32).reshape(NW, EPW)
    dst = edge_index[1].astype(jnp.int32).reshape(NW, EPW)
    srcp = jnp.pad(src, ((0, 0), (0, EPAD - EPW))).reshape(NW, NCHUNK, CH)
    dstp = jnp.pad(dst, ((0, 0), (0, EPAD - EPW))).reshape(NW, NCHUNK, CH)
    att2 = jnp.stack([att_src, att_dst], axis=1)
    h, a_s, a_d = _pre(x, W, att2)
    num, den = _sc_edges(h, a_s.reshape(N), a_d.reshape(N), srcp, dstp)
    den_t = den.T
    batch3 = batch.astype(jnp.int32).reshape(NBLK, 1, BLK)
    return _post(num, den_t, batch3, bias.reshape(1, D), Wf, bf.reshape(1, 1))


# speedup vs baseline: 16.5728x; 16.5728x over previous
import jax
import jax.numpy as jnp
from jax import lax
from jax.experimental import pallas as pl
from jax.experimental.pallas import tpu as pltpu
from jax.experimental.pallas import tpu_sc as plsc

N = 10000
E = 320000
D = 128
DH = D // 2
G = 128
NC, NS, L = 2, 16, 16
EPT = E // NS
CH = 128
NCHUNK = (EPT + CH - 1) // CH
EPAD = NCHUNK * CH
BLK = 1000
NBLK = N // BLK
NZCH = (N + CH - 1) // CH


def _pre_body(x_ref, w_ref, att2_ref, h2_ref, as_ref, ad_ref):
    h = jnp.dot(x_ref[...], w_ref[...], preferred_element_type=jnp.float32)
    h2_ref[0] = h[:, :DH]
    h2_ref[1] = h[:, DH:]
    asd = jnp.dot(h, att2_ref[...], preferred_element_type=jnp.float32)
    as_ref[...] = asd[:, 0:1]
    ad_ref[...] = asd[:, 1:2]


_pre = pl.pallas_call(
    _pre_body,
    grid=(NBLK,),
    in_specs=[
        pl.BlockSpec((BLK, D), lambda i: (i, 0)),
        pl.BlockSpec((D, D), lambda i: (0, 0)),
        pl.BlockSpec((D, 2), lambda i: (0, 0)),
    ],
    out_specs=[
        pl.BlockSpec((NC, BLK, DH), lambda i: (0, i, 0)),
        pl.BlockSpec((BLK, 1), lambda i: (i, 0)),
        pl.BlockSpec((BLK, 1), lambda i: (i, 0)),
    ],
    out_shape=[
        jax.ShapeDtypeStruct((NC, N, DH), jnp.float32),
        jax.ShapeDtypeStruct((N, 1), jnp.float32),
        jax.ShapeDtypeStruct((N, 1), jnp.float32),
    ],
)


def _sc_body(h2_hbm, as_hbm, ad_hbm, src_hbm, dst_hbm, zeros_hbm,
             num_out, den_out,
             src_v, dst_v, as_v, ad_v, ex_v, rows_v,
             num_sh, den_sh, sem):
    cid = lax.axis_index("c")
    sid = lax.axis_index("s")

    for k in range(5):
        c = sid + k * NS

        @pl.when(c < NZCH - 1)
        def _():
            pltpu.sync_copy(zeros_hbm, num_sh.at[pl.ds(c * CH, CH)])

        @pl.when(c == NZCH - 1)
        def _():
            rem = N - (NZCH - 1) * CH
            pltpu.sync_copy(zeros_hbm.at[pl.ds(0, rem)],
                            num_sh.at[pl.ds((NZCH - 1) * CH, rem)])
    NDCH = N // DH
    for k in range(10):
        c = sid + k * NS

        @pl.when(c < NDCH)
        def _():
            pltpu.sync_copy(zeros_hbm.at[0], den_sh.at[pl.ds(c * DH, DH)])

        @pl.when(c == NDCH)
        def _():
            pltpu.sync_copy(zeros_hbm.at[0, pl.ds(0, N - NDCH * DH)],
                            den_sh.at[pl.ds(NDCH * DH, N - NDCH * DH)])
    plsc.subcore_barrier()

    pltpu.sync_copy(src_hbm.at[sid], src_v)
    pltpu.sync_copy(dst_hbm.at[sid], dst_v)
    pltpu.sync_copy(as_hbm, as_v)
    pltpu.sync_copy(ad_hbm, ad_v)

    iota16 = lax.iota(jnp.int32, L)
    h_half = h2_hbm.at[cid]

    def _chunk(c, carry):
        desc = pltpu.async_copy(h_half.at[src_v.at[c]], rows_v, sem)
        for j in range(CH // L):
            s16 = src_v[c, pl.ds(j * L, L)]
            d16 = dst_v[c, pl.ds(j * L, L)]
            a_s = plsc.load_gather(as_v, [s16])
            a_d = plsc.load_gather(ad_v, [d16])
            e = a_s + a_d
            e = jnp.where(e > 0.0, e, 0.2 * e)
            ex = jnp.exp(e)
            valid = (c * CH + j * L + iota16) < EPT
            ex_v[c, pl.ds(j * L, L)] = jnp.where(valid, ex, 0.0)
        pltpu.sync_copy(ex_v.at[c], den_sh.at[dst_v.at[c]], add=True)
        desc.wait()

        def _sgrp(jj, carry2):
            exv = ex_v[c, pl.ds(jj * L, L)]
            for rl in range(L):
                r = jj * L + rl
                b = jnp.full((L,), exv[rl], jnp.float32)
                for k in range(DH // L):
                    rows_v[r, pl.ds(k * L, L)] = rows_v[r, pl.ds(k * L, L)] * b
            return carry2
        lax.fori_loop(0, CH // L, _sgrp, 0)
        pltpu.sync_copy(rows_v, num_sh.at[dst_v.at[c]], add=True)
        return carry
    lax.fori_loop(0, NCHUNK, _chunk, 0)

    plsc.subcore_barrier()

    rpt = 624
    pltpu.sync_copy(num_sh.at[pl.ds(sid * rpt, rpt)],
                    num_out.at[cid, pl.ds(sid * rpt, rpt)])

    @pl.when(sid == NS - 1)
    def _():
        pltpu.sync_copy(num_sh.at[pl.ds(NS * rpt, N - NS * rpt)],
                        num_out.at[cid, pl.ds(NS * rpt, N - NS * rpt)])

    @pl.when(sid == 0)
    def _():
        pltpu.sync_copy(den_sh, den_out.at[cid])


_sc_edges = pl.kernel(
    _sc_body,
    out_type=[
        jax.ShapeDtypeStruct((NC, N, DH), jnp.float32),
        jax.ShapeDtypeStruct((NC, N), jnp.float32),
    ],
    mesh=plsc.VectorSubcoreMesh(core_axis_name="c", subcore_axis_name="s"),
    compiler_params=pltpu.CompilerParams(needs_layout_passes=False,
                                         use_tc_tiling_on_sc=False),
    scratch_types=[
        pltpu.VMEM((NCHUNK, CH), jnp.int32),
        pltpu.VMEM((NCHUNK, CH), jnp.int32),
        pltpu.VMEM((N,), jnp.float32),
        pltpu.VMEM((N,), jnp.float32),
        pltpu.VMEM((NCHUNK, CH), jnp.float32),
        pltpu.VMEM((CH, DH), jnp.float32),
        pltpu.VMEM_SHARED((N, DH), jnp.float32),
        pltpu.VMEM_SHARED((N,), jnp.float32),
        pltpu.SemaphoreType.DMA,
    ],
)


def _post_body(num_ref, den_ref, batch_ref, bias_ref, wf_ref, bf_ref,
               out_ref, sums_ref, counts_ref):
    i = pl.program_id(0)

    @pl.when(i == 0)
    def _():
        sums_ref[...] = jnp.zeros_like(sums_ref)
        counts_ref[...] = jnp.zeros_like(counts_ref)

    halves = []
    for c in range(NC):
        den = den_ref[...][:, c]
        safe = jnp.where(den > 0.0, den, 1.0)
        halves.append(jnp.where(den[:, None] > 0.0,
                                num_ref[c] / safe[:, None], 0.0))
    out = jnp.concatenate(halves, axis=1) + bias_ref[...]
    g = out * jax.nn.sigmoid(out)
    b = batch_ref[0, 0, :]
    cols = lax.broadcasted_iota(jnp.int32, (BLK, G), 1)
    p = (b[:, None] == cols).astype(jnp.float32)
    sums_ref[...] += lax.dot_general(
        p, g, (((0,), (0,)), ((), ())), preferred_element_type=jnp.float32)
    counts_ref[...] += lax.dot_general(
        p, jnp.ones((BLK, 1), jnp.float32), (((0,), (0,)), ((), ())),
        preferred_element_type=jnp.float32)

    @pl.when(i == NBLK - 1)
    def _():
        mean = sums_ref[...] / jnp.maximum(counts_ref[...], 1.0)
        out_ref[...] = jnp.dot(mean, wf_ref[...],
                               preferred_element_type=jnp.float32) + bf_ref[...]


_post = pl.pallas_call(
    _post_body,
    grid=(NBLK,),
    in_specs=[
        pl.BlockSpec((NC, BLK, DH), lambda i: (0, i, 0)),
        pl.BlockSpec((BLK, NC), lambda i: (i, 0)),
        pl.BlockSpec((1, 1, BLK), lambda i: (i, 0, 0)),
        pl.BlockSpec((1, D), lambda i: (0, 0)),
        pl.BlockSpec((D, 1), lambda i: (0, 0)),
        pl.BlockSpec((1, 1), lambda i: (0, 0)),
    ],
    out_specs=pl.BlockSpec((G, 1), lambda i: (0, 0)),
    out_shape=jax.ShapeDtypeStruct((G, 1), jnp.float32),
    scratch_shapes=[
        pltpu.VMEM((G, D), jnp.float32),
        pltpu.VMEM((G, 1), jnp.float32),
    ],
)


def kernel(x, edge_index, batch, W, att_src, att_dst, bias, Wf, bf):
    src = edge_index[0].astype(jnp.int32).reshape(NS, EPT)
    dst = edge_index[1].astype(jnp.int32).reshape(NS, EPT)
    srcp = jnp.pad(src, ((0, 0), (0, EPAD - EPT))).reshape(NS, NCHUNK, CH)
    dstp = jnp.pad(dst, ((0, 0), (0, EPAD - EPT))).reshape(NS, NCHUNK, CH)
    att2 = jnp.stack([att_src, att_dst], axis=1)
    h2, a_s, a_d = _pre(x, W, att2)
    zeros_blk = jnp.zeros((CH, DH), jnp.float32)
    num, den = _sc_edges(h2, a_s.reshape(N), a_d.reshape(N), srcp, dstp,
                         zeros_blk)
    den_t = den.T
    batch3 = batch.astype(jnp.int32).reshape(NBLK, 1, BLK)
    return _post(num, den_t, batch3, bias.reshape(1, D), Wf, bf.reshape(1, 1))


# speedup vs baseline: 16.5844x; 1.0007x over previous
import jax
import jax.numpy as jnp
from jax import lax
from jax.experimental import pallas as pl
from jax.experimental.pallas import tpu as pltpu
from jax.experimental.pallas import tpu_sc as plsc

N = 10000
E = 320000
D = 128
DH = D // 2
G = 128
NC, NS, L = 2, 16, 16
EPT = E // NS
CH = 128
NCHUNK = (EPT + CH - 1) // CH
EPAD = NCHUNK * CH
BLK = 1000
NBLK = N // BLK
NZCH = (N + CH - 1) // CH


def _pre_body(x_ref, w_ref, att2_ref, h2_ref, as_ref, ad_ref):
    h = jnp.dot(x_ref[...], w_ref[...], preferred_element_type=jnp.float32)
    h2_ref[0] = h[:, :DH]
    h2_ref[1] = h[:, DH:]
    asd = jnp.dot(h, att2_ref[...], preferred_element_type=jnp.float32)
    as_ref[...] = asd[:, 0:1]
    ad_ref[...] = asd[:, 1:2]


_pre = pl.pallas_call(
    _pre_body,
    grid=(NBLK,),
    in_specs=[
        pl.BlockSpec((BLK, D), lambda i: (i, 0)),
        pl.BlockSpec((D, D), lambda i: (0, 0)),
        pl.BlockSpec((D, 2), lambda i: (0, 0)),
    ],
    out_specs=[
        pl.BlockSpec((NC, BLK, DH), lambda i: (0, i, 0)),
        pl.BlockSpec((BLK, 1), lambda i: (i, 0)),
        pl.BlockSpec((BLK, 1), lambda i: (i, 0)),
    ],
    out_shape=[
        jax.ShapeDtypeStruct((NC, N, DH), jnp.float32),
        jax.ShapeDtypeStruct((N, 1), jnp.float32),
        jax.ShapeDtypeStruct((N, 1), jnp.float32),
    ],
)


def _sc_body(h2_hbm, as_hbm, ad_hbm, src_hbm, dst_hbm, zeros_hbm,
             num_out, den_out,
             src_v, dst_v, as_v, ad_v, ex_v, rows0_v,
             num_sh, den_sh, gsem0, dsem0):
    cid = lax.axis_index("c")
    sid = lax.axis_index("s")

    for k in range(5):
        c = sid + k * NS

        @pl.when(c < NZCH - 1)
        def _():
            pltpu.sync_copy(zeros_hbm, num_sh.at[pl.ds(c * CH, CH)])

        @pl.when(c == NZCH - 1)
        def _():
            rem = N - (NZCH - 1) * CH
            pltpu.sync_copy(zeros_hbm.at[pl.ds(0, rem)],
                            num_sh.at[pl.ds((NZCH - 1) * CH, rem)])
    NDCH = N // DH
    for k in range(10):
        c = sid + k * NS

        @pl.when(c < NDCH)
        def _():
            pltpu.sync_copy(zeros_hbm.at[0], den_sh.at[pl.ds(c * DH, DH)])

        @pl.when(c == NDCH)
        def _():
            pltpu.sync_copy(zeros_hbm.at[0, pl.ds(0, N - NDCH * DH)],
                            den_sh.at[pl.ds(NDCH * DH, N - NDCH * DH)])
    plsc.subcore_barrier()

    pltpu.sync_copy(src_hbm.at[sid], src_v)
    pltpu.sync_copy(dst_hbm.at[sid], dst_v)
    pltpu.sync_copy(as_hbm, as_v)
    pltpu.sync_copy(ad_hbm, ad_v)

    iota16 = lax.iota(jnp.int32, L)
    h_half = h2_hbm.at[cid]

    def _weights(c):
        for j in range(CH // L):
            s16 = src_v[c, pl.ds(j * L, L)]
            d16 = dst_v[c, pl.ds(j * L, L)]
            a_s = plsc.load_gather(as_v, [s16])
            a_d = plsc.load_gather(ad_v, [d16])
            e = a_s + a_d
            e = jnp.where(e > 0.0, e, 0.2 * e)
            ex = jnp.exp(e)
            valid = (c * CH + j * L + iota16) < EPT
            ex_v[c, pl.ds(j * L, L)] = jnp.where(valid, ex, 0.0)

    def _scale(c, rbuf):
        def _sgrp(jj, carry2):
            exv = ex_v[c, pl.ds(jj * L, L)]
            for rl in range(L):
                r = jj * L + rl
                b = jnp.full((L,), exv[rl], jnp.float32)
                for k in range(DH // L):
                    rbuf[r, pl.ds(k * L, L)] = rbuf[r, pl.ds(k * L, L)] * b
            return carry2
        lax.fori_loop(0, CH // L, _sgrp, 0)

    def _chunk(c, carry):
        g0 = pltpu.async_copy(h_half.at[src_v.at[c]], rows0_v, gsem0)
        _weights(c)
        d0 = pltpu.async_copy(ex_v.at[c], den_sh.at[dst_v.at[c]],
                              dsem0, add=True)
        g0.wait()
        _scale(c, rows0_v)
        pltpu.sync_copy(rows0_v, num_sh.at[dst_v.at[c]], add=True)
        d0.wait()
        return carry
    lax.fori_loop(0, NCHUNK, _chunk, 0)

    plsc.subcore_barrier()

    rpt = 624
    pltpu.sync_copy(num_sh.at[pl.ds(sid * rpt, rpt)],
                    num_out.at[cid, pl.ds(sid * rpt, rpt)])

    @pl.when(sid == NS - 1)
    def _():
        pltpu.sync_copy(num_sh.at[pl.ds(NS * rpt, N - NS * rpt)],
                        num_out.at[cid, pl.ds(NS * rpt, N - NS * rpt)])

    @pl.when(sid == 0)
    def _():
        pltpu.sync_copy(den_sh, den_out.at[cid])


_sc_edges = pl.kernel(
    _sc_body,
    out_type=[
        jax.ShapeDtypeStruct((NC, N, DH), jnp.float32),
        jax.ShapeDtypeStruct((NC, N), jnp.float32),
    ],
    mesh=plsc.VectorSubcoreMesh(core_axis_name="c", subcore_axis_name="s"),
    compiler_params=pltpu.CompilerParams(needs_layout_passes=False,
                                         use_tc_tiling_on_sc=False),
    scratch_types=[
        pltpu.VMEM((NCHUNK, CH), jnp.int32),
        pltpu.VMEM((NCHUNK, CH), jnp.int32),
        pltpu.VMEM((N,), jnp.float32),
        pltpu.VMEM((N,), jnp.float32),
        pltpu.VMEM((NCHUNK, CH), jnp.float32),
        pltpu.VMEM((CH, DH), jnp.float32),
        pltpu.VMEM_SHARED((N, DH), jnp.float32),
        pltpu.VMEM_SHARED((N,), jnp.float32),
        pltpu.SemaphoreType.DMA,
        pltpu.SemaphoreType.DMA,
    ],
)


def _post_body(num_ref, den_ref, batch_ref, bias_ref, wf_ref, bf_ref,
               out_ref, sums_ref, counts_ref):
    i = pl.program_id(0)

    @pl.when(i == 0)
    def _():
        sums_ref[...] = jnp.zeros_like(sums_ref)
        counts_ref[...] = jnp.zeros_like(counts_ref)

    halves = []
    for c in range(NC):
        den = den_ref[...][:, c]
        safe = jnp.where(den > 0.0, den, 1.0)
        halves.append(jnp.where(den[:, None] > 0.0,
                                num_ref[c] / safe[:, None], 0.0))
    out = jnp.concatenate(halves, axis=1) + bias_ref[...]
    g = out * jax.nn.sigmoid(out)
    b = batch_ref[0, 0, :]
    cols = lax.broadcasted_iota(jnp.int32, (BLK, G), 1)
    p = (b[:, None] == cols).astype(jnp.float32)
    sums_ref[...] += lax.dot_general(
        p, g, (((0,), (0,)), ((), ())), preferred_element_type=jnp.float32)
    counts_ref[...] += lax.dot_general(
        p, jnp.ones((BLK, 1), jnp.float32), (((0,), (0,)), ((), ())),
        preferred_element_type=jnp.float32)

    @pl.when(i == NBLK - 1)
    def _():
        mean = sums_ref[...] / jnp.maximum(counts_ref[...], 1.0)
        out_ref[...] = jnp.dot(mean, wf_ref[...],
                               preferred_element_type=jnp.float32) + bf_ref[...]


_post = pl.pallas_call(
    _post_body,
    grid=(NBLK,),
    in_specs=[
        pl.BlockSpec((NC, BLK, DH), lambda i: (0, i, 0)),
        pl.BlockSpec((BLK, NC), lambda i: (i, 0)),
        pl.BlockSpec((1, 1, BLK), lambda i: (i, 0, 0)),
        pl.BlockSpec((1, D), lambda i: (0, 0)),
        pl.BlockSpec((D, 1), lambda i: (0, 0)),
        pl.BlockSpec((1, 1), lambda i: (0, 0)),
    ],
    out_specs=pl.BlockSpec((G, 1), lambda i: (0, 0)),
    out_shape=jax.ShapeDtypeStruct((G, 1), jnp.float32),
    scratch_shapes=[
        pltpu.VMEM((G, D), jnp.float32),
        pltpu.VMEM((G, 1), jnp.float32),
    ],
)


def kernel(x, edge_index, batch, W, att_src, att_dst, bias, Wf, bf):
    src = edge_index[0].astype(jnp.int32).reshape(NS, EPT)
    dst = edge_index[1].astype(jnp.int32).reshape(NS, EPT)
    srcp = jnp.pad(src, ((0, 0), (0, EPAD - EPT))).reshape(NS, NCHUNK, CH)
    dstp = jnp.pad(dst, ((0, 0), (0, EPAD - EPT))).reshape(NS, NCHUNK, CH)
    att2 = jnp.stack([att_src, att_dst], axis=1)
    h2, a_s, a_d = _pre(x, W, att2)
    zeros_blk = jnp.zeros((CH, DH), jnp.float32)
    num, den = _sc_edges(h2, a_s.reshape(N), a_d.reshape(N), srcp, dstp,
                         zeros_blk)
    den_t = den.T
    batch3 = batch.astype(jnp.int32).reshape(NBLK, 1, BLK)
    return _post(num, den_t, batch3, bias.reshape(1, D), Wf,
                 bf.reshape(1, 1))


# speedup vs baseline: 27.3248x; 1.6476x over previous
import jax
import jax.numpy as jnp
from jax import lax
from jax.experimental import pallas as pl
from jax.experimental.pallas import tpu as pltpu
from jax.experimental.pallas import tpu_sc as plsc

N = 10000
E = 320000
D = 128
DH = D // 2
G = 128
NC, NS, L = 2, 16, 16
EPT = E // NS
CH = 128
NCHUNK = 158
EPAD = NCHUNK * CH
BLK = 1000
NBLK = N // BLK
NZCH = (N + CH - 1) // CH


def _pre_body(x_ref, w_ref, att2_ref, h2_ref, as_ref, ad_ref):
    h = jnp.dot(x_ref[...], w_ref[...], preferred_element_type=jnp.float32)
    h2_ref[0] = h[:, :DH]
    h2_ref[1] = h[:, DH:]
    asd = jnp.dot(h, att2_ref[...], preferred_element_type=jnp.float32)
    as_ref[...] = asd[:, 0:1]
    ad_ref[...] = asd[:, 1:2]


_pre = pl.pallas_call(
    _pre_body,
    grid=(NBLK,),
    in_specs=[
        pl.BlockSpec((BLK, D), lambda i: (i, 0)),
        pl.BlockSpec((D, D), lambda i: (0, 0)),
        pl.BlockSpec((D, 2), lambda i: (0, 0)),
    ],
    out_specs=[
        pl.BlockSpec((NC, BLK, DH), lambda i: (0, i, 0)),
        pl.BlockSpec((BLK, 1), lambda i: (i, 0)),
        pl.BlockSpec((BLK, 1), lambda i: (i, 0)),
    ],
    out_shape=[
        jax.ShapeDtypeStruct((NC, N, DH), jnp.float32),
        jax.ShapeDtypeStruct((N, 1), jnp.float32),
        jax.ShapeDtypeStruct((N, 1), jnp.float32),
    ],
)


def _sc_body(h2_hbm, as_hbm, ad_hbm, src_hbm, dst_hbm, zeros_hbm,
             num_out, den_out,
             src_v, dst_v, as_v, ad_v, ex0_v, ex1_v, rows0_v, rows1_v,
             num_sh, den_sh, gsem0, gsem1, ssem0, ssem1, dsem0, dsem1):
    cid = lax.axis_index("c")
    sid = lax.axis_index("s")

    for k in range(5):
        c = sid + k * NS

        @pl.when(c < NZCH - 1)
        def _():
            pltpu.sync_copy(zeros_hbm, num_sh.at[pl.ds(c * CH, CH)])

        @pl.when(c == NZCH - 1)
        def _():
            rem = N - (NZCH - 1) * CH
            pltpu.sync_copy(zeros_hbm.at[pl.ds(0, rem)],
                            num_sh.at[pl.ds((NZCH - 1) * CH, rem)])
    NDCH = N // DH
    for k in range(10):
        c = sid + k * NS

        @pl.when(c < NDCH)
        def _():
            pltpu.sync_copy(zeros_hbm.at[0], den_sh.at[pl.ds(c * DH, DH)])

        @pl.when(c == NDCH)
        def _():
            pltpu.sync_copy(zeros_hbm.at[0, pl.ds(0, N - NDCH * DH)],
                            den_sh.at[pl.ds(NDCH * DH, N - NDCH * DH)])
    plsc.subcore_barrier()

    pltpu.sync_copy(src_hbm.at[sid], src_v)
    pltpu.sync_copy(dst_hbm.at[sid], dst_v)
    pltpu.sync_copy(as_hbm, as_v)
    pltpu.sync_copy(ad_hbm, ad_v)

    iota16 = lax.iota(jnp.int32, L)
    h_half = h2_hbm.at[cid]

    def _weights(c, exbuf):
        for j in range(CH // L):
            s16 = src_v[c, pl.ds(j * L, L)]
            d16 = dst_v[c, pl.ds(j * L, L)]
            a_s = plsc.load_gather(as_v, [s16])
            a_d = plsc.load_gather(ad_v, [d16])
            e = a_s + a_d
            e = jnp.where(e > 0.0, e, 0.2 * e)
            ex = jnp.exp(e)
            valid = (c * CH + j * L + iota16) < EPT
            exbuf[pl.ds(j * L, L)] = jnp.where(valid, ex, 0.0)

    def _scale(exbuf, rbuf):
        def _sgrp(jj, carry2):
            exv = exbuf[pl.ds(jj * L, L)]
            for rl in range(L):
                r = jj * L + rl
                b = jnp.full((L,), exv[rl], jnp.float32)
                for k in range(DH // L):
                    rbuf[r, pl.ds(k * L, L)] = rbuf[r, pl.ds(k * L, L)] * b
            return carry2
        lax.fori_loop(0, CH // L, _sgrp, 0)

    def _duo(i, carry):
        c0 = 2 * i
        c1 = c0 + 1
        g0 = pltpu.async_copy(h_half.at[src_v.at[c0]], rows0_v, gsem0)
        g1 = pltpu.async_copy(h_half.at[src_v.at[c1]], rows1_v, gsem1)
        _weights(c0, ex0_v)
        d0 = pltpu.async_copy(ex0_v, den_sh.at[dst_v.at[c0]],
                              dsem0, add=True)
        _weights(c1, ex1_v)
        d1 = pltpu.async_copy(ex1_v, den_sh.at[dst_v.at[c1]],
                              dsem1, add=True)
        g0.wait()
        _scale(ex0_v, rows0_v)
        s0 = pltpu.async_copy(rows0_v, num_sh.at[dst_v.at[c0]],
                              ssem0, add=True)
        g1.wait()
        _scale(ex1_v, rows1_v)
        s1 = pltpu.async_copy(rows1_v, num_sh.at[dst_v.at[c1]],
                              ssem1, add=True)
        d0.wait()
        d1.wait()
        s0.wait()
        s1.wait()
        return carry
    lax.fori_loop(0, NCHUNK // 2, _duo, 0)

    plsc.subcore_barrier()

    rpt = 624
    pltpu.sync_copy(num_sh.at[pl.ds(sid * rpt, rpt)],
                    num_out.at[cid, pl.ds(sid * rpt, rpt)])

    @pl.when(sid == NS - 1)
    def _():
        pltpu.sync_copy(num_sh.at[pl.ds(NS * rpt, N - NS * rpt)],
                        num_out.at[cid, pl.ds(NS * rpt, N - NS * rpt)])

    @pl.when(sid == 0)
    def _():
        pltpu.sync_copy(den_sh, den_out.at[cid])


_sc_edges = pl.kernel(
    _sc_body,
    out_type=[
        jax.ShapeDtypeStruct((NC, N, DH), jnp.float32),
        jax.ShapeDtypeStruct((NC, N), jnp.float32),
    ],
    mesh=plsc.VectorSubcoreMesh(core_axis_name="c", subcore_axis_name="s"),
    compiler_params=pltpu.CompilerParams(needs_layout_passes=False,
                                         use_tc_tiling_on_sc=False),
    scratch_types=[
        pltpu.VMEM((NCHUNK, CH), jnp.int32),
        pltpu.VMEM((NCHUNK, CH), jnp.int32),
        pltpu.VMEM((N,), jnp.float32),
        pltpu.VMEM((N,), jnp.float32),
        pltpu.VMEM((CH,), jnp.float32),
        pltpu.VMEM((CH,), jnp.float32),
        pltpu.VMEM((CH, DH), jnp.float32),
        pltpu.VMEM((CH, DH), jnp.float32),
        pltpu.VMEM_SHARED((N, DH), jnp.float32),
        pltpu.VMEM_SHARED((N,), jnp.float32),
        pltpu.SemaphoreType.DMA,
        pltpu.SemaphoreType.DMA,
        pltpu.SemaphoreType.DMA,
        pltpu.SemaphoreType.DMA,
        pltpu.SemaphoreType.DMA,
        pltpu.SemaphoreType.DMA,
    ],
)


def _post_body(num_ref, den_ref, batch_ref, bias_ref, wf_ref, bf_ref,
               out_ref, sums_ref, counts_ref):
    i = pl.program_id(0)

    @pl.when(i == 0)
    def _():
        sums_ref[...] = jnp.zeros_like(sums_ref)
        counts_ref[...] = jnp.zeros_like(counts_ref)

    halves = []
    for c in range(NC):
        den = den_ref[...][:, c]
        safe = jnp.where(den > 0.0, den, 1.0)
        halves.append(jnp.where(den[:, None] > 0.0,
                                num_ref[c] / safe[:, None], 0.0))
    out = jnp.concatenate(halves, axis=1) + bias_ref[...]
    g = out * jax.nn.sigmoid(out)
    b = batch_ref[0, 0, :]
    cols = lax.broadcasted_iota(jnp.int32, (BLK, G), 1)
    p = (b[:, None] == cols).astype(jnp.float32)
    sums_ref[...] += lax.dot_general(
        p, g, (((0,), (0,)), ((), ())), preferred_element_type=jnp.float32)
    counts_ref[...] += lax.dot_general(
        p, jnp.ones((BLK, 1), jnp.float32), (((0,), (0,)), ((), ())),
        preferred_element_type=jnp.float32)

    @pl.when(i == NBLK - 1)
    def _():
        mean = sums_ref[...] / jnp.maximum(counts_ref[...], 1.0)
        out_ref[...] = jnp.dot(mean, wf_ref[...],
                               preferred_element_type=jnp.float32) + bf_ref[...]


_post = pl.pallas_call(
    _post_body,
    grid=(NBLK,),
    in_specs=[
        pl.BlockSpec((NC, BLK, DH), lambda i: (0, i, 0)),
        pl.BlockSpec((BLK, NC), lambda i: (i, 0)),
        pl.BlockSpec((1, 1, BLK), lambda i: (i, 0, 0)),
        pl.BlockSpec((1, D), lambda i: (0, 0)),
        pl.BlockSpec((D, 1), lambda i: (0, 0)),
        pl.BlockSpec((1, 1), lambda i: (0, 0)),
    ],
    out_specs=pl.BlockSpec((G, 1), lambda i: (0, 0)),
    out_shape=jax.ShapeDtypeStruct((G, 1), jnp.float32),
    scratch_shapes=[
        pltpu.VMEM((G, D), jnp.float32),
        pltpu.VMEM((G, 1), jnp.float32),
    ],
)


def kernel(x, edge_index, batch, W, att_src, att_dst, bias, Wf, bf):
    src = edge_index[0].astype(jnp.int32).reshape(NS, EPT)
    dst = edge_index[1].astype(jnp.int32).reshape(NS, EPT)
    srcp = jnp.pad(src, ((0, 0), (0, EPAD - EPT))).reshape(NS, NCHUNK, CH)
    dstp = jnp.pad(dst, ((0, 0), (0, EPAD - EPT))).reshape(NS, NCHUNK, CH)
    att2 = jnp.stack([att_src, att_dst], axis=1)
    h2, a_s, a_d = _pre(x, W, att2)
    zeros_blk = jnp.zeros((CH, DH), jnp.float32)
    num, den = _sc_edges(h2, a_s.reshape(N), a_d.reshape(N), srcp, dstp,
                         zeros_blk)
    den_t = den.T
    batch3 = batch.astype(jnp.int32).reshape(NBLK, 1, BLK)
    return _post(num, den_t, batch3, bias.reshape(1, D), Wf,
                 bf.reshape(1, 1))
